# baseline (speedup 1.0000x reference)
import functools

import jax
import jax.numpy as jnp
from jax import lax
from jax.experimental import pallas as pl
from jax.experimental.pallas import tpu as pltpu
from jax.experimental.pallas import tpu_sc as plsc

_C = 768
_J = 224 * 224
_NC, _NS = 2, 16
_NW = _NC * _NS
_JPW = _J // _NW
_B = 32
_NCHUNK = _JPW // _B
_G = _C // 16


def _sc_body(x_hbm, idx_hbm, out_hbm, idx_v, inbuf, outbuf, gsem, ssem):
    wid = lax.axis_index("s") * _NC + lax.axis_index("c")
    base = wid * (_JPW * 2)
    pltpu.sync_copy(idx_hbm, idx_v)

    def in_copy(k, b):
        return pltpu.make_async_copy(
            x_hbm.at[pl.ds(base + k * _B, _B)], inbuf.at[b], gsem.at[b]
        )

    def out_copy(k, b):
        return pltpu.make_async_copy(
            inbuf.at[b], out_hbm.at[pl.ds(base + k * _B, _B)], ssem.at[b]
        )

    cvecs = [idx_v[pl.ds(g * 16, 16)] for g in range(_G)]

    def make_row_body(b):
        def row_body(j, c2):
            jvec = jnp.full((16,), j, dtype=jnp.int32)

            def loads(g0):
                return [
                    plsc.load_gather(inbuf.at[b], [jvec, cvecs[g]])
                    for g in range(g0, g0 + 8)
                ]

            def stores(g0, vs):
                for g, v in zip(range(g0, g0 + 8), vs):
                    outbuf[b, j, pl.ds(g * 16, 16)] = v

            prev = loads(0)
            for g0 in range(8, _G, 8):
                cur = loads(g0)
                stores(g0 - 8, prev)
                prev = cur
            stores(_G - 8, prev)
            return c2

        return row_body

    row_bodies = [make_row_body(0), make_row_body(1)]

    NCH = _NCHUNK * 2

    @pl.when(wid < 16)
    def _():
        in_copy(0, 0).start()

        def pair_body(p, carry):
            for sub in range(2):
                k = 2 * p + sub

                @pl.when(k < NCH)
                def _():
                    in_copy(k, sub).wait()

                    @pl.when(k + 1 < NCH)
                    def _():
                        in_copy(k + 1, 1 - sub).start()

                    @pl.when(k >= 2)
                    def _():
                        out_copy(k - 2, sub).wait()

                    out_copy(k, sub).start()

            return carry

        lax.fori_loop(0, (NCH + 1) // 2, pair_body, 0)
        out_copy(NCH - 2, (NCH - 2) % 2).wait()
        out_copy(NCH - 1, (NCH - 1) % 2).wait()


@jax.jit
def _sc_permute(x2, idx):
    mesh = plsc.VectorSubcoreMesh(
        core_axis_name="c", subcore_axis_name="s", num_cores=_NC, num_subcores=_NS
    )
    return pl.kernel(
        _sc_body,
        out_type=jax.ShapeDtypeStruct((_J, _C), jnp.float32),
        mesh=mesh,
        compiler_params=pltpu.CompilerParams(needs_layout_passes=False),
        scratch_types=[
            pltpu.VMEM((_C,), jnp.int32),
            pltpu.VMEM((2, _B, _C), jnp.float32),
            pltpu.VMEM((2, _B, _C), jnp.float32),
            pltpu.SemaphoreType.DMA((2,)),
            pltpu.SemaphoreType.DMA((2,)),
        ],
    )(x2, idx)


def kernel(inp):
    C, H, W = inp.shape
    perm = jax.random.permutation(jax.random.key(42), C).astype(jnp.int32)
    x2 = jnp.transpose(inp, (1, 2, 0)).reshape(H * W, C)
    y2 = _sc_permute(x2, perm)
    return jnp.transpose(y2.reshape(H, W, C), (2, 0, 1))


# speedup vs baseline: 1.5713x; 1.5713x over previous
import jax
import jax.numpy as jnp
import numpy as np
from jax import lax
from jax.experimental import pallas as pl
from jax.experimental.pallas import tpu as pltpu
from jax.experimental.pallas import tpu_sc as plsc

_C = 768
_PERM = np.asarray(jax.random.permutation(jax.random.key(42), _C)).astype(np.int32)
_J = 224 * 224
_NC, _NS = 2, 16
_NW = _NC * _NS
_JPW = _J // _NW
_B = 32
_NCHUNK = _JPW // _B
_G = _C // 16


def _sc_body(x_hbm, idx_hbm, out_hbm, idx_v, inbuf, outbuf, gsem, ssem):
    wid = lax.axis_index("s") * _NC + lax.axis_index("c")
    base = wid * _JPW
    pltpu.sync_copy(idx_hbm, idx_v)

    def in_copy(k, b):
        return pltpu.make_async_copy(
            x_hbm.at[pl.ds(base + k * _B, _B)], inbuf.at[b], gsem.at[b]
        )

    def out_copy(k, b):
        return pltpu.make_async_copy(
            outbuf.at[b], out_hbm.at[pl.ds(base + k * _B, _B)], ssem.at[b]
        )

    cvecs = [idx_v[pl.ds(g * 16, 16)] for g in range(_G)]

    def make_row_body(b):
        def row_body(j, c2):
            jvec = jnp.full((16,), j, dtype=jnp.int32)

            def loads(g0):
                return [
                    plsc.load_gather(inbuf.at[b], [jvec, cvecs[g]])
                    for g in range(g0, g0 + 8)
                ]

            def stores(g0, vs):
                for g, v in zip(range(g0, g0 + 8), vs):
                    outbuf[b, j, pl.ds(g * 16, 16)] = v

            prev = loads(0)
            for g0 in range(8, _G, 8):
                cur = loads(g0)
                stores(g0 - 8, prev)
                prev = cur
            stores(_G - 8, prev)
            return c2

        return row_body

    row_bodies = [make_row_body(0), make_row_body(1)]

    in_copy(0, 0).start()

    def pair_body(p, carry):
        for sub in range(2):
            k = 2 * p + sub

            @pl.when(k < _NCHUNK)
            def _():
                in_copy(k, sub).wait()

                @pl.when(k + 1 < _NCHUNK)
                def _():
                    in_copy(k + 1, 1 - sub).start()

                @pl.when(k >= 2)
                def _():
                    out_copy(k - 2, sub).wait()

                lax.fori_loop(0, _B, row_bodies[sub], 0)
                out_copy(k, sub).start()

        return carry

    lax.fori_loop(0, (_NCHUNK + 1) // 2, pair_body, 0)
    out_copy(_NCHUNK - 2, (_NCHUNK - 2) % 2).wait()
    out_copy(_NCHUNK - 1, (_NCHUNK - 1) % 2).wait()


@jax.jit
def _sc_permute(x2, idx):
    mesh = plsc.VectorSubcoreMesh(
        core_axis_name="c", subcore_axis_name="s", num_cores=_NC, num_subcores=_NS
    )
    return pl.kernel(
        _sc_body,
        out_type=jax.ShapeDtypeStruct((_J, _C), jnp.float32),
        mesh=mesh,
        compiler_params=pltpu.CompilerParams(needs_layout_passes=False),
        scratch_types=[
            pltpu.VMEM((_C,), jnp.int32),
            pltpu.VMEM((2, _B, _C), jnp.float32),
            pltpu.VMEM((2, _B, _C), jnp.float32),
            pltpu.SemaphoreType.DMA((2,)),
            pltpu.SemaphoreType.DMA((2,)),
        ],
    )(x2, idx)


def kernel(inp):
    C, H, W = inp.shape
    perm = jnp.asarray(_PERM)
    x2 = jnp.transpose(inp, (1, 2, 0)).reshape(H * W, C)
    y2 = _sc_permute(x2, perm)
    return jnp.transpose(y2.reshape(H, W, C), (2, 0, 1))
